# speedup vs baseline: 1.5737x; 1.5737x over previous
import functools

import jax
import jax.numpy as jnp
from jax import lax
from jax.experimental import pallas as pl
from jax.experimental.pallas import tpu as pltpu
from jax.experimental.pallas import tpu_sc as plsc

B, N, D = 4, 4096, 1024

NC, NS = 2, 16
NW = NC * NS
ROWS_PER_W = N // NW
CH = 64
NCHUNK = ROWS_PER_W // CH

_mesh = plsc.VectorSubcoreMesh(core_axis_name="c", subcore_axis_name="s")


@functools.partial(
    pl.kernel,
    out_type=jax.ShapeDtypeStruct((B, N, D), jnp.float32),
    mesh=_mesh,
    scratch_types=[
        pltpu.VMEM((CH, D), jnp.float32),
        pltpu.SemaphoreType.DMA,
    ],
)
def _pos_embed_sc(table_hbm, out_hbm, buf, sem):
    wid = lax.axis_index("s") * NC + lax.axis_index("c")
    for g in range(NCHUNK):
        base = wid * ROWS_PER_W + g * CH
        pltpu.sync_copy(table_hbm.at[pl.ds(base, CH)], buf)
        copies = [
            pltpu.async_copy(buf, out_hbm.at[b, pl.ds(base, CH)], sem)
            for b in range(B)
        ]
        for c in copies:
            c.wait()


def kernel(patches, table):
    del patches
    return _pos_embed_sc(table)
